# baseline (speedup 1.0000x reference)
import numpy as np
import jax
import jax.numpy as jnp
from jax.experimental import pallas as pl

_D_IN = 16
_N = 26
_SIZES = [_N - i for i in range(_N)]
_TOTAL = sum(_SIZES)
_OFFS = np.cumsum([0] + _SIZES)
_PAD = 128

_BB = 256


def _body(x_ref, w_ref, b_ref, *out_refs):
    x = x_ref[...]
    for i in range(_N):
        y = jax.lax.dot_general(
            x, w_ref[i], (((1,), (0,)), ((), ())),
            preferred_element_type=jnp.float32)
        y = y + b_ref[i]
        out_refs[i][...] = y[:, : _SIZES[i]]


def kernel(x, W_cat, b_cat):
    B = x.shape[0]
    Wt = W_cat.T
    heads_w = [
        jnp.pad(Wt[:, _OFFS[i]:_OFFS[i + 1]], ((0, 0), (0, _PAD - _SIZES[i])))
        for i in range(_N)
    ]
    W_heads = jnp.stack(heads_w)
    heads_b = [
        jnp.pad(b_cat[_OFFS[i]:_OFFS[i + 1]], (0, _PAD - _SIZES[i]))
        for i in range(_N)
    ]
    b_heads = jnp.stack(heads_b)[:, None, :]

    grid = (B // _BB,)
    out_shapes = [
        jax.ShapeDtypeStruct((B, _SIZES[i]), jnp.float32) for i in range(_N)
    ]
    out_specs = [
        pl.BlockSpec((_BB, _SIZES[i]), lambda i: (i, 0)) for i in range(_N)
    ]
    in_specs = [
        pl.BlockSpec((_BB, _D_IN), lambda i: (i, 0)),
        pl.BlockSpec((_N, _D_IN, _PAD), lambda i: (0, 0, 0)),
        pl.BlockSpec((_N, 1, _PAD), lambda i: (0, 0, 0)),
    ]
    outs = pl.pallas_call(
        _body,
        grid=grid,
        in_specs=in_specs,
        out_specs=out_specs,
        out_shape=out_shapes,
    )(x, W_heads, b_heads)
    return tuple(outs)


# speedup vs baseline: 1.0528x; 1.0528x over previous
import numpy as np
import jax
import jax.numpy as jnp
from jax.experimental import pallas as pl

_D_IN = 16
_N = 26
_SIZES = [_N - i for i in range(_N)]
_TOTAL = sum(_SIZES)
_OFFS = np.cumsum([0] + _SIZES)
_PAD = 128

_BB = 1024


def _body(x_ref, w_ref, b_ref, *out_refs):
    x = x_ref[...]
    for i in range(_N):
        y = jax.lax.dot_general(
            x, w_ref[i], (((1,), (0,)), ((), ())),
            preferred_element_type=jnp.float32)
        y = y + b_ref[i]
        out_refs[i][...] = y[:, : _SIZES[i]]


def kernel(x, W_cat, b_cat):
    B = x.shape[0]
    Wt = W_cat.T
    heads_w = [
        jnp.pad(Wt[:, _OFFS[i]:_OFFS[i + 1]], ((0, 0), (0, _PAD - _SIZES[i])))
        for i in range(_N)
    ]
    W_heads = jnp.stack(heads_w)
    heads_b = [
        jnp.pad(b_cat[_OFFS[i]:_OFFS[i + 1]], (0, _PAD - _SIZES[i]))
        for i in range(_N)
    ]
    b_heads = jnp.stack(heads_b)[:, None, :]

    grid = (B // _BB,)
    out_shapes = [
        jax.ShapeDtypeStruct((B, _SIZES[i]), jnp.float32) for i in range(_N)
    ]
    out_specs = [
        pl.BlockSpec((_BB, _SIZES[i]), lambda i: (i, 0)) for i in range(_N)
    ]
    in_specs = [
        pl.BlockSpec((_BB, _D_IN), lambda i: (i, 0)),
        pl.BlockSpec((_N, _D_IN, _PAD), lambda i: (0, 0, 0)),
        pl.BlockSpec((_N, 1, _PAD), lambda i: (0, 0, 0)),
    ]
    outs = pl.pallas_call(
        _body,
        grid=grid,
        in_specs=in_specs,
        out_specs=out_specs,
        out_shape=out_shapes,
    )(x, W_heads, b_heads)
    return tuple(outs)
